# initial kernel scaffold (unmeasured)
import jax
import jax.numpy as jnp
from jax import lax
from jax.experimental import pallas as pl
from jax.experimental.pallas import tpu as pltpu

N_DEV = 4
WINDOW = 128


def kernel(x, Wq, K_ext, V_ext, Wo):
    B, Sq, C = x.shape
    _, Skv, Hq, Dh = K_ext.shape
    HD = Hq * Dh
    Co = Wo.shape[1]

    K2 = K_ext.reshape(B, Skv, HD)
    V2 = V_ext.reshape(B, Skv, HD)

    def body(x_ref, wq_ref, k_ref, v_ref, wo_ref, out_ref,
             q_ref, ctx_ref, comm_u, comm_s, su, ru, ss, rs):
        my = lax.axis_index("i")
        left = lax.rem(my + N_DEV - 1, N_DEV)
        right = lax.rem(my + 1, N_DEV)

        barrier_sem = pltpu.get_barrier_semaphore()
        for nbr in [left, right]:
            pl.semaphore_signal(
                barrier_sem, inc=1,
                device_id=(nbr,), device_id_type=pl.DeviceIdType.MESH,
            )
        pl.semaphore_wait(barrier_sem, 2)

        for b in range(B):
            q_ref[b] = jnp.dot(
                x_ref[b], wq_ref[...], preferred_element_type=jnp.float32
            )

        qi = lax.broadcasted_iota(jnp.int32, (Sq, Skv), 0)
        kj = lax.broadcasted_iota(jnp.int32, (Sq, Skv), 1) + my * Skv
        mask = jnp.abs(qi - kj) <= WINDOW
        for b in range(B):
            for h in range(Hq):
                qbh = q_ref[b, :, h * Dh:(h + 1) * Dh]
                kbh = k_ref[b, :, h * Dh:(h + 1) * Dh]
                vbh = v_ref[b, :, h * Dh:(h + 1) * Dh]
                s = lax.dot_general(
                    qbh, kbh, (((1,), (1,)), ((), ())),
                    preferred_element_type=jnp.float32,
                ) * 0.125
                s = jnp.where(mask, s, -1e9)
                m = jnp.max(s, axis=1, keepdims=True)
                w = jnp.exp(s - m)
                keep = m > -1e8
                w = jnp.where(keep, w, 0.0)
                l = jnp.sum(w, axis=1, keepdims=True)
                u = jnp.dot(w, vbh, preferred_element_type=jnp.float32)
                comm_u[0, b, :, h * Dh:(h + 1) * Dh] = u
                comm_s[0, b, :, 2 * h:2 * h + 1] = l
                comm_s[0, b, :, 2 * h + 1:2 * h + 2] = m

        for h in range(N_DEV - 1):
            rdma_u = pltpu.make_async_remote_copy(
                src_ref=comm_u.at[h], dst_ref=comm_u.at[h + 1],
                send_sem=su.at[h], recv_sem=ru.at[h],
                device_id=(right,), device_id_type=pl.DeviceIdType.MESH,
            )
            rdma_s = pltpu.make_async_remote_copy(
                src_ref=comm_s.at[h], dst_ref=comm_s.at[h + 1],
                send_sem=ss.at[h], recv_sem=rs.at[h],
                device_id=(right,), device_id_type=pl.DeviceIdType.MESH,
            )
            rdma_u.start()
            rdma_s.start()
            rdma_u.wait()
            rdma_s.wait()

        for b in range(B):
            for h in range(Hq):
                m = comm_s[0, b, :, 2 * h + 1:2 * h + 2]
                for sl in range(1, N_DEV):
                    m = jnp.maximum(m, comm_s[sl, b, :, 2 * h + 1:2 * h + 2])
                l_tot = jnp.zeros((Sq, 1), jnp.float32)
                u_tot = jnp.zeros((Sq, Dh), jnp.float32)
                for sl in range(N_DEV):
                    sc = jnp.exp(comm_s[sl, b, :, 2 * h + 1:2 * h + 2] - m)
                    l_tot += comm_s[sl, b, :, 2 * h:2 * h + 1] * sc
                    u_tot += comm_u[sl, b, :, h * Dh:(h + 1) * Dh] * sc
                ctx_ref[b, :, h * Dh:(h + 1) * Dh] = u_tot / l_tot

        for b in range(B):
            out_ref[b] = jnp.dot(
                ctx_ref[b], wo_ref[...], preferred_element_type=jnp.float32
            )

    return pl.pallas_call(
        body,
        out_shape=jax.ShapeDtypeStruct((B, Sq, Co), jnp.float32),
        in_specs=[pl.BlockSpec(memory_space=pltpu.VMEM)] * 5,
        out_specs=pl.BlockSpec(memory_space=pltpu.VMEM),
        scratch_shapes=[
            pltpu.VMEM((B, Sq, HD), jnp.float32),
            pltpu.VMEM((B, Sq, HD), jnp.float32),
            pltpu.VMEM((N_DEV, B, Sq, HD), jnp.float32),
            pltpu.VMEM((N_DEV, B, Sq, 2 * Hq), jnp.float32),
            pltpu.SemaphoreType.DMA((N_DEV - 1,)),
            pltpu.SemaphoreType.DMA((N_DEV - 1,)),
            pltpu.SemaphoreType.DMA((N_DEV - 1,)),
            pltpu.SemaphoreType.DMA((N_DEV - 1,)),
        ],
        compiler_params=pltpu.CompilerParams(collective_id=0),
    )(x, Wq, K2, V2, Wo)


# baseline (device time: 131589 ns/iter reference)
import jax
import jax.numpy as jnp
from jax import lax
from jax.experimental import pallas as pl
from jax.experimental.pallas import tpu as pltpu

N_DEV = 4
WINDOW = 128


def kernel(x, Wq, K_ext, V_ext, Wo):
    B, Sq, C = x.shape
    _, Skv, Hq, Dh = K_ext.shape
    HD = Hq * Dh
    Co = Wo.shape[1]

    K2 = K_ext.reshape(B, Skv, HD)
    V2 = V_ext.reshape(B, Skv, HD)

    def body(x_ref, wq_ref, k_ref, v_ref, wo_ref, out_ref,
             q_ref, ctx_ref, comm_u, comm_s, su, ru, ss, rs):
        my = lax.axis_index("i")
        left = lax.rem(my + N_DEV - 1, N_DEV)
        right = lax.rem(my + 1, N_DEV)

        barrier_sem = pltpu.get_barrier_semaphore()
        for nbr in [left, right]:
            pl.semaphore_signal(
                barrier_sem, inc=1,
                device_id=(nbr,), device_id_type=pl.DeviceIdType.MESH,
            )
        pl.semaphore_wait(barrier_sem, 2)

        for b in range(B):
            q_ref[b] = jnp.dot(
                x_ref[b], wq_ref[...], preferred_element_type=jnp.float32
            )

        qi = lax.broadcasted_iota(jnp.int32, (Sq, Skv), 0)
        kj = lax.broadcasted_iota(jnp.int32, (Sq, Skv), 1) + my * Skv
        mask = jnp.abs(qi - kj) <= WINDOW
        for b in range(B):
            for h in range(Hq):
                qbh = q_ref[b, :, h * Dh:(h + 1) * Dh]
                kbh = k_ref[b, :, h * Dh:(h + 1) * Dh]
                vbh = v_ref[b, :, h * Dh:(h + 1) * Dh]
                s = lax.dot_general(
                    qbh, kbh, (((1,), (1,)), ((), ())),
                    preferred_element_type=jnp.float32,
                ) * 0.125
                s = jnp.where(mask, s, -1e9)
                m = jnp.max(s, axis=1, keepdims=True)
                w = jnp.exp(s - m)
                keep = m > -1e8
                w = jnp.where(keep, w, 0.0)
                l = jnp.sum(w, axis=1, keepdims=True)
                u = jnp.dot(w, vbh, preferred_element_type=jnp.float32)
                comm_u[0, b, :, h * Dh:(h + 1) * Dh] = u
                comm_s[0, b, :, 2 * h:2 * h + 1] = l
                comm_s[0, b, :, 2 * h + 1:2 * h + 2] = m

        for h in range(N_DEV - 1):
            rdma_u = pltpu.make_async_remote_copy(
                src_ref=comm_u.at[h], dst_ref=comm_u.at[h + 1],
                send_sem=su.at[h], recv_sem=ru.at[h],
                device_id=(right,), device_id_type=pl.DeviceIdType.MESH,
            )
            rdma_s = pltpu.make_async_remote_copy(
                src_ref=comm_s.at[h], dst_ref=comm_s.at[h + 1],
                send_sem=ss.at[h], recv_sem=rs.at[h],
                device_id=(right,), device_id_type=pl.DeviceIdType.MESH,
            )
            rdma_u.start()
            rdma_s.start()
            rdma_u.wait()
            rdma_s.wait()

        for b in range(B):
            for h in range(Hq):
                m = comm_s[0, b, :, 2 * h + 1:2 * h + 2]
                for sl in range(1, N_DEV):
                    m = jnp.maximum(m, comm_s[sl, b, :, 2 * h + 1:2 * h + 2])
                l_tot = jnp.zeros((Sq, 1), jnp.float32)
                u_tot = jnp.zeros((Sq, Dh), jnp.float32)
                for sl in range(N_DEV):
                    sc = jnp.exp(comm_s[sl, b, :, 2 * h + 1:2 * h + 2] - m)
                    l_tot += comm_s[sl, b, :, 2 * h:2 * h + 1] * sc
                    u_tot += comm_u[sl, b, :, h * Dh:(h + 1) * Dh] * sc
                ctx_ref[b, :, h * Dh:(h + 1) * Dh] = u_tot / l_tot

        for b in range(B):
            out_ref[b] = jnp.dot(
                ctx_ref[b], wo_ref[...], preferred_element_type=jnp.float32
            )

    return pl.pallas_call(
        body,
        out_shape=jax.ShapeDtypeStruct((B, Sq, Co), jnp.float32),
        in_specs=[pl.BlockSpec(memory_space=pltpu.VMEM)] * 5,
        out_specs=pl.BlockSpec(memory_space=pltpu.VMEM),
        scratch_shapes=[
            pltpu.VMEM((B, Sq, HD), jnp.float32),
            pltpu.VMEM((B, Sq, HD), jnp.float32),
            pltpu.VMEM((N_DEV, B, Sq, HD), jnp.float32),
            pltpu.VMEM((N_DEV, B, Sq, 2 * Hq), jnp.float32),
            pltpu.SemaphoreType.DMA((N_DEV - 1,)),
            pltpu.SemaphoreType.DMA((N_DEV - 1,)),
            pltpu.SemaphoreType.DMA((N_DEV - 1,)),
            pltpu.SemaphoreType.DMA((N_DEV - 1,)),
        ],
        compiler_params=pltpu.CompilerParams(
            collective_id=0, vmem_limit_bytes=100 * 1024 * 1024
        ),
    )(x, Wq, K2, V2, Wo)
